# baseline (speedup 1.0000x reference)
import functools

import jax
import jax.numpy as jnp
from jax import lax
from jax.experimental import pallas as pl
from jax.experimental.pallas import tpu as pltpu
from jax.experimental.pallas import tpu_sc as plsc

EMBED_DIM = 32
NUM_CORES = 2
NUM_SUBCORES = 16
NUM_WORKERS = NUM_CORES * NUM_SUBCORES
LANES = 16
BCHUNK = 16


@functools.lru_cache(maxsize=None)
def _build(batch: int, hist: int, vocab_rows: int):
    per_worker = batch // NUM_WORKERS
    num_chunks = per_worker // BCHUNK
    assert batch % NUM_WORKERS == 0 and per_worker % BCHUNK == 0
    full_heads = [h0 for h0 in range(0, hist - LANES + 1, LANES)]
    tail_head = hist - LANES
    mesh = plsc.VectorSubcoreMesh(core_axis_name="c", subcore_axis_name="s")

    @functools.partial(
        pl.kernel,
        mesh=mesh,
        compiler_params=pltpu.CompilerParams(
            use_tc_tiling_on_sc=False, needs_layout_passes=False),
        out_type=jax.ShapeDtypeStruct((batch, hist, EMBED_DIM), jnp.float32),
        scratch_types=[
            pltpu.VMEM((vocab_rows, EMBED_DIM), jnp.float32),
            pltpu.VMEM((2, BCHUNK, hist), jnp.int32),
            pltpu.VMEM((2, BCHUNK, hist, EMBED_DIM), jnp.float32),
            pltpu.SemaphoreType.DMA((2,)),
            pltpu.SemaphoreType.DMA((2,)),
        ],
    )
    def gather_kernel(idx_hbm, table_hbm, out_hbm, table_v, idx_v, rows_v,
                      isem, osem):
        wid = lax.axis_index("s") * NUM_CORES + lax.axis_index("c")
        base = wid * per_worker

        pltpu.sync_copy(table_hbm, table_v)
        pltpu.async_copy(idx_hbm.at[pl.ds(base, BCHUNK)], idx_v.at[0],
                         isem.at[0])

        def copy_row(buf, e, h0, lanes, vec):
            src = vec + 1
            for l in lanes:
                b = src[l]
                rows_v[buf, e, h0 + l, pl.ds(0, LANES)] = (
                    table_v[b, pl.ds(0, LANES)])
                rows_v[buf, e, h0 + l, pl.ds(LANES, LANES)] = (
                    table_v[b, pl.ds(LANES, LANES)])

        def chunk_body(k, carry):
            buf = lax.rem(k, 2)
            nbuf = 1 - buf

            @pl.when(k + 1 < num_chunks)
            def _prefetch():
                pltpu.async_copy(
                    idx_hbm.at[pl.ds(base + (k + 1) * BCHUNK, BCHUNK)],
                    idx_v.at[nbuf], isem.at[nbuf])

            pltpu.make_async_copy(
                idx_hbm.at[pl.ds(base + k * BCHUNK, BCHUNK)],
                idx_v.at[buf], isem.at[buf]).wait()

            @pl.when(k >= 2)
            def _drain():
                pltpu.make_async_copy(
                    rows_v.at[buf],
                    out_hbm.at[pl.ds(base + (k - 2) * BCHUNK, BCHUNK)],
                    osem.at[buf]).wait()

            @plsc.parallel_loop(0, BCHUNK, unroll=1)
            def entry_body(e):
                for h0 in full_heads:
                    vec = idx_v[buf, e, pl.ds(h0, LANES)]
                    copy_row(buf, e, h0, range(LANES), vec)
                if tail_head not in full_heads:
                    done = full_heads[-1] + LANES - tail_head
                    vec = idx_v[buf, e, pl.ds(tail_head, LANES)]
                    copy_row(buf, e, tail_head, range(done, LANES), vec)

            pltpu.async_copy(
                rows_v.at[buf],
                out_hbm.at[pl.ds(base + k * BCHUNK, BCHUNK)],
                osem.at[buf])
            return carry

        lax.fori_loop(0, num_chunks, chunk_body, 0)

        for k in (num_chunks - 2, num_chunks - 1):
            pltpu.make_async_copy(
                rows_v.at[k % 2],
                out_hbm.at[pl.ds(base + k * BCHUNK, BCHUNK)],
                osem.at[k % 2]).wait()

    return gather_kernel


def kernel(indices, table):
    batch, hist = indices.shape
    return _build(batch, hist, table.shape[0])(indices, table)


# speedup vs baseline: 6.8652x; 6.8652x over previous
import functools

import jax
import jax.numpy as jnp
from jax import lax
from jax.experimental import pallas as pl
from jax.experimental.pallas import tpu as pltpu
from jax.experimental.pallas import tpu_sc as plsc

EMBED_DIM = 32
PAD_ROW = EMBED_DIM + 1
NUM_CORES = 2
NUM_SUBCORES = 16
NUM_WORKERS = NUM_CORES * NUM_SUBCORES
LANES = 16
SUBLANES = 8
LANES128 = 128


@functools.lru_cache(maxsize=None)
def _build(batch: int, hist: int, vocab_rows: int):
    b_per_w = batch // NUM_WORKERS
    bt_per_w = b_per_w // LANES128
    groups = b_per_w // LANES
    d_tiles = EMBED_DIM // SUBLANES
    assert batch % (NUM_WORKERS * LANES128) == 0
    mesh = plsc.VectorSubcoreMesh(core_axis_name="c", subcore_axis_name="s")

    @functools.partial(
        pl.kernel,
        mesh=mesh,
        compiler_params=pltpu.CompilerParams(
            use_tc_tiling_on_sc=False, needs_layout_passes=False),
        out_type=jax.ShapeDtypeStruct(
            (hist, d_tiles, batch // LANES128, SUBLANES, LANES128),
            jnp.float32),
        scratch_types=[
            pltpu.VMEM((vocab_rows * PAD_ROW,), jnp.float32),
            pltpu.VMEM((hist, b_per_w), jnp.int32),
            pltpu.VMEM((2, d_tiles, bt_per_w, SUBLANES, LANES128),
                       jnp.float32),
            pltpu.SemaphoreType.DMA((2,)),
        ],
    )
    def gather_kernel(idxt_hbm, table_hbm, out_hbm, table_v, islab, stg,
                      osem):
        wid = lax.axis_index("s") * NUM_CORES + lax.axis_index("c")
        b0 = wid * b_per_w
        bt0 = wid * bt_per_w

        pltpu.sync_copy(table_hbm, table_v)
        pltpu.sync_copy(idxt_hbm.at[:, pl.ds(b0, b_per_w)], islab)

        def step_body(h, carry):
            buf = lax.rem(h, 2)

            @pl.when(h >= 2)
            def _drain():
                pltpu.make_async_copy(
                    stg.at[buf], out_hbm.at[h - 2, :, pl.ds(bt0, bt_per_w)],
                    osem.at[buf]).wait()

            @plsc.parallel_loop(0, groups, unroll=1)
            def group_body(g):
                vec = islab[h, pl.ds(g * LANES, LANES)]
                addr = (vec + 1) * PAD_ROW
                bt = g // (LANES128 // LANES)
                bs0 = lax.rem(g, LANES128 // LANES) * LANES
                for dt in range(d_tiles):
                    for ds in range(SUBLANES):
                        col = plsc.load_gather(
                            table_v, [addr + (dt * SUBLANES + ds)])
                        stg[buf, dt, bt, ds, pl.ds(bs0, LANES)] = col

            pltpu.async_copy(
                stg.at[buf], out_hbm.at[h, :, pl.ds(bt0, bt_per_w)],
                osem.at[buf])
            return carry

        lax.fori_loop(0, hist, step_body, 0)

        for h in (hist - 2, hist - 1):
            pltpu.make_async_copy(
                stg.at[h % 2], out_hbm.at[h, :, pl.ds(bt0, bt_per_w)],
                osem.at[h % 2]).wait()

    return gather_kernel


def kernel(indices, table):
    batch, hist = indices.shape
    idx_t = indices.T
    table_pad = jnp.pad(table, ((0, 0), (0, PAD_ROW - EMBED_DIM)))
    table_flat = table_pad.reshape(table.shape[0] * PAD_ROW)
    out5 = _build(batch, hist, table.shape[0])(idx_t, table_flat)
    return out5.transpose((2, 4, 0, 1, 3)).reshape(batch, hist, EMBED_DIM)
